# Initial kernel scaffold; baseline (speedup 1.0000x reference)
import jax
import jax.numpy as jnp
from jax.experimental import pallas as pl


def kernel(x, table):
    raise NotImplementedError("write your pallas kernel here")


# speedup vs baseline: 3.9564x; 3.9564x over previous
import functools

import jax
import jax.numpy as jnp
from jax import lax
from jax.experimental import pallas as pl
from jax.experimental.pallas import tpu as pltpu
from jax.experimental.pallas import tpu_sc as plsc

HIDDEN = 64
NC, NS = 2, 16
NW = NC * NS
N_TOKENS = 4096 * 200
B_PER_W = N_TOKENS // NW
CHUNK = 512
NCHUNK = B_PER_W // CHUNK

_mesh = plsc.VectorSubcoreMesh(core_axis_name="c", subcore_axis_name="s")


@functools.partial(
    pl.kernel,
    mesh=_mesh,
    compiler_params=pltpu.CompilerParams(use_tc_tiling_on_sc=False),
    out_type=jax.ShapeDtypeStruct((N_TOKENS, HIDDEN), jnp.float32),
    scratch_types=[
        pltpu.VMEM((CHUNK,), jnp.int32),
        pltpu.VMEM((CHUNK, HIDDEN), jnp.float32),
        pltpu.SemaphoreType.DMA,
    ],
)
def _embed_gather(x_hbm, table_hbm, out_hbm, idx_v, rows_v, sem):
    wid = lax.axis_index("s") * NC + lax.axis_index("c")
    base0 = wid * B_PER_W

    def body(i, _):
        base = base0 + i * CHUNK
        pltpu.sync_copy(x_hbm.at[pl.ds(base, CHUNK)], idx_v)
        pltpu.async_copy(table_hbm.at[idx_v], rows_v, sem).wait()
        pltpu.sync_copy(rows_v, out_hbm.at[pl.ds(base, CHUNK)])
        return 0

    lax.fori_loop(0, NCHUNK, body, 0)


def kernel(x, table):
    xf = x.reshape(-1).astype(jnp.int32)
    out = _embed_gather(xf, table)
    return out.reshape(x.shape + (HIDDEN,))


# speedup vs baseline: 4.2691x; 1.0790x over previous
import functools

import jax
import jax.numpy as jnp
from jax import lax
from jax.experimental import pallas as pl
from jax.experimental.pallas import tpu as pltpu
from jax.experimental.pallas import tpu_sc as plsc

HIDDEN = 64
NC, NS = 2, 16
NW = NC * NS
N_TOKENS = 4096 * 200
B_PER_W = N_TOKENS // NW
CHUNK = 512
NCHUNK = B_PER_W // CHUNK
NBUF = 2

_mesh = plsc.VectorSubcoreMesh(core_axis_name="c", subcore_axis_name="s")


@functools.partial(
    pl.kernel,
    mesh=_mesh,
    compiler_params=pltpu.CompilerParams(use_tc_tiling_on_sc=False),
    out_type=jax.ShapeDtypeStruct((N_TOKENS, HIDDEN), jnp.float32),
    scratch_types=[
        pltpu.VMEM((B_PER_W,), jnp.int32),
        pltpu.VMEM((NBUF, CHUNK, HIDDEN), jnp.float32),
        pltpu.SemaphoreType.DMA((NBUF,)),
        pltpu.SemaphoreType.DMA((NBUF,)),
    ],
)
def _embed_gather(x_hbm, table_hbm, out_hbm, idx_all, rows_v, sem_g, sem_o):
    wid = lax.axis_index("s") * NC + lax.axis_index("c")
    base0 = wid * B_PER_W
    pltpu.sync_copy(x_hbm.at[pl.ds(base0, B_PER_W)], idx_all)

    def gather_desc(i, b):
        idx = idx_all.at[pl.ds(i * CHUNK, CHUNK)]
        return pltpu.make_async_copy(table_hbm.at[idx], rows_v.at[b], sem_g.at[b])

    def out_desc(i, b):
        dst = out_hbm.at[pl.ds(base0 + i * CHUNK, CHUNK)]
        return pltpu.make_async_copy(rows_v.at[b], dst, sem_o.at[b])

    gather_desc(0, 0).start()
    gather_desc(1, 1).start()
    gather_desc(0, 0).wait()
    out_desc(0, 0).start()

    def group(g, _):
        for b in range(NBUF):
            i = g * NBUF + b
            out_desc(i - NBUF, b).wait()
            gather_desc(i, b).start()
            gather_desc(i - 1, 1 - b).wait()
            out_desc(i - 1, 1 - b).start()
        return 0

    lax.fori_loop(1, NCHUNK // NBUF, group, 0)

    gather_desc(NCHUNK - 1, 1).wait()
    out_desc(NCHUNK - 1, 1).start()
    out_desc(NCHUNK - 2, 0).wait()
    out_desc(NCHUNK - 1, 1).wait()


def kernel(x, table):
    xf = x.reshape(-1).astype(jnp.int32)
    out = _embed_gather(xf, table)
    return out.reshape(x.shape + (HIDDEN,))
